# baseline (speedup 1.0000x reference)
import functools

import jax
import jax.numpy as jnp
from jax import lax
from jax.experimental import pallas as pl
from jax.experimental.pallas import tpu as pltpu
from jax.experimental.pallas import tpu_sc as plsc

E = 320000
D = 128
NC, NS, L = 2, 16, 16
NW = NC * NS
EPW = E // NW
CB = 80
NCHUNK = EPW // CB
GB = CB // L

_mesh = plsc.VectorSubcoreMesh(core_axis_name="c", subcore_axis_name="s")


@functools.partial(
    pl.kernel,
    mesh=_mesh,
    compiler_params=pltpu.CompilerParams(needs_layout_passes=False),
    out_type=jax.ShapeDtypeStruct((E,), jnp.float32),
    scratch_types=[
        pltpu.VMEM((EPW,), jnp.int32),
        pltpu.VMEM((EPW,), jnp.int32),
        pltpu.VMEM((CB, D), jnp.float32),
        pltpu.VMEM((CB, D), jnp.float32),
        pltpu.VMEM((CB, D), jnp.float32),
        pltpu.VMEM((CB, D), jnp.float32),
        pltpu.VMEM((EPW,), jnp.float32),
        pltpu.SemaphoreType.DMA,
        pltpu.SemaphoreType.DMA,
    ],
)
def _ipd(z_hbm, src_hbm, dst_hbm, out_hbm,
         si_v, di_v, sr0, sr1, dr0, dr1, out_v, sem0, sem1):
    wid = lax.axis_index("s") * NC + lax.axis_index("c")
    base = wid * EPW
    sbufs, dbufs, sems = (sr0, sr1), (dr0, dr1), (sem0, sem1)

    pltpu.sync_copy(src_hbm.at[pl.ds(base, EPW)], si_v)
    pltpu.sync_copy(dst_hbm.at[pl.ds(base, EPW)], di_v)

    def start(b, c):
        pltpu.async_copy(z_hbm.at[si_v.at[pl.ds(c * CB, CB)]], sbufs[b], sems[b])
        pltpu.async_copy(z_hbm.at[di_v.at[pl.ds(c * CB, CB)]], dbufs[b], sems[b])

    def drain(b):
        dummy = z_hbm.at[pl.ds(0, CB)]
        pltpu.make_async_copy(dummy, sbufs[b], sems[b]).wait()
        pltpu.make_async_copy(dummy, dbufs[b], sems[b]).wait()

    def compute(b, c):
        srows_v, drows_v = sbufs[b], dbufs[b]

        def group_body(g, gcarry):
            lane = lax.broadcasted_iota(jnp.int32, (L,), 0)
            rows = g * L + lane

            def k_body(k, acc):
                cols = (lane + k) & (D - 1)
                s = plsc.load_gather(srows_v, [rows, cols])
                d = plsc.load_gather(drows_v, [rows, cols])
                return acc + s * d

            acc = lax.fori_loop(0, D, k_body, jnp.zeros((L,), jnp.float32),
                                unroll=32)
            out_v[pl.ds(c * CB + g * L, L)] = 1.0 / (1.0 + jnp.exp(-acc))
            return gcarry

        lax.fori_loop(0, GB, group_body, 0)

    start(0, 0)
    start(1, 1)

    def chunk_body(c, carry):
        for b in (0, 1):
            @pl.when(c % 2 == b)
            def _():
                drain(b)
                compute(b, c)

                @pl.when(c + 2 < NCHUNK)
                def _():
                    start(b, c + 2)

        return carry

    lax.fori_loop(0, NCHUNK, chunk_body, 0)
    pltpu.sync_copy(out_v, out_hbm.at[pl.ds(base, EPW)])


def kernel(z, edge_index):
    ei = edge_index.astype(jnp.int32)
    return _ipd(z, ei[0], ei[1])


# speedup vs baseline: 1.2429x; 1.2429x over previous
import functools

import jax
import jax.numpy as jnp
from jax import lax
from jax.experimental import pallas as pl
from jax.experimental.pallas import tpu as pltpu
from jax.experimental.pallas import tpu_sc as plsc

E = 320000
D = 128
NC, NS, L = 2, 16, 16
NW = NC * NS
EPW = E // NW
CB = 80
NCHUNK = EPW // CB
GB = CB // L
NBUF = 4

_mesh = plsc.VectorSubcoreMesh(core_axis_name="c", subcore_axis_name="s")


@functools.partial(
    pl.kernel,
    mesh=_mesh,
    compiler_params=pltpu.CompilerParams(needs_layout_passes=False),
    out_type=jax.ShapeDtypeStruct((E,), jnp.float32),
    scratch_types=[
        pltpu.VMEM((EPW,), jnp.int32),
        pltpu.VMEM((EPW,), jnp.int32),
        pltpu.VMEM((CB, D), jnp.float32),
        pltpu.VMEM((CB, D), jnp.float32),
        pltpu.VMEM((CB, D), jnp.float32),
        pltpu.VMEM((CB, D), jnp.float32),
        pltpu.VMEM((CB, D), jnp.float32),
        pltpu.VMEM((CB, D), jnp.float32),
        pltpu.VMEM((CB, D), jnp.float32),
        pltpu.VMEM((CB, D), jnp.float32),
        pltpu.VMEM((EPW,), jnp.float32),
        pltpu.SemaphoreType.DMA,
        pltpu.SemaphoreType.DMA,
        pltpu.SemaphoreType.DMA,
        pltpu.SemaphoreType.DMA,
    ],
)
def _ipd(z_hbm, src_hbm, dst_hbm, out_hbm,
         si_v, di_v, sr0, sr1, sr2, sr3, dr0, dr1, dr2, dr3, out_v,
         sem0, sem1, sem2, sem3):
    wid = lax.axis_index("s") * NC + lax.axis_index("c")
    base = wid * EPW
    sbufs, dbufs = (sr0, sr1, sr2, sr3), (dr0, dr1, dr2, dr3)
    sems = (sem0, sem1, sem2, sem3)

    pltpu.sync_copy(src_hbm.at[pl.ds(base, EPW)], si_v)
    pltpu.sync_copy(dst_hbm.at[pl.ds(base, EPW)], di_v)

    def start(b, c):
        pltpu.async_copy(z_hbm.at[si_v.at[pl.ds(c * CB, CB)]], sbufs[b], sems[b])
        pltpu.async_copy(z_hbm.at[di_v.at[pl.ds(c * CB, CB)]], dbufs[b], sems[b])

    def drain(b):
        dummy = z_hbm.at[pl.ds(0, CB)]
        pltpu.make_async_copy(dummy, sbufs[b], sems[b]).wait()
        pltpu.make_async_copy(dummy, dbufs[b], sems[b]).wait()

    def compute(b, c):
        srows_v, drows_v = sbufs[b], dbufs[b]

        def group_body(g, gcarry):
            lane = lax.broadcasted_iota(jnp.int32, (L,), 0)
            rows = g * L + lane

            def k_body(k, acc):
                cols = (lane + k) & (D - 1)
                s = plsc.load_gather(srows_v, [rows, cols])
                d = plsc.load_gather(drows_v, [rows, cols])
                return acc + s * d

            acc = lax.fori_loop(0, D, k_body, jnp.zeros((L,), jnp.float32),
                                unroll=32)
            out_v[pl.ds(c * CB + g * L, L)] = 1.0 / (1.0 + jnp.exp(-acc))
            return gcarry

        lax.fori_loop(0, GB, group_body, 0)

    for b in range(NBUF):
        start(b, b)

    def chunk_body(c, carry):
        for b in range(NBUF):
            @pl.when(c % NBUF == b)
            def _():
                drain(b)
                compute(b, c)

                @pl.when(c + NBUF < NCHUNK)
                def _():
                    start(b, c + NBUF)

        return carry

    lax.fori_loop(0, NCHUNK, chunk_body, 0)
    pltpu.sync_copy(out_v, out_hbm.at[pl.ds(base, EPW)])


def kernel(z, edge_index):
    ei = edge_index.astype(jnp.int32)
    return _ipd(z, ei[0], ei[1])
